# baseline (speedup 1.0000x reference)
import functools

import jax
import jax.numpy as jnp
from jax import lax
from jax.experimental import pallas as pl
from jax.experimental.pallas import tpu as pltpu
from jax.experimental.pallas import tpu_sc as plsc

N = 10000
E = 320000
D_IN = 128
D_HID = 64
D_OUT = 16

NW = 32
CHUNK = 128
NCH = E // CHUNK
RPW = NCH // NW
NEXTRA = NCH - NW * RPW
BUFR = 3
NITER = RPW // BUFR
NROWS = 10240
ROWS_PER_TILE = NROWS // 16
DEG_W = 4

_mesh = plsc.VectorSubcoreMesh(core_axis_name="c", subcore_axis_name="s")
_sc_params = pltpu.CompilerParams(
    use_tc_tiling_on_sc=False, needs_layout_passes=False)


def _wid():
    return lax.axis_index("s") * 2 + lax.axis_index("c")


@functools.partial(
    pl.kernel,
    out_type=jax.ShapeDtypeStruct((2, NROWS, DEG_W), jnp.float32),
    mesh=_mesh,
    compiler_params=_sc_params,
    scratch_types=[
        pltpu.VMEM((RPW + 1, CHUNK), jnp.int32),
        pltpu.VMEM((CHUNK, DEG_W), jnp.float32),
        pltpu.VMEM_SHARED((NROWS, DEG_W), jnp.float32),
    ],
)
def _sc_degree(dst_hbm, ones_hbm, zeros_hbm, out_hbm, dst_v, ones_v, deg_sh):
    c = lax.axis_index("c")
    s = lax.axis_index("s")
    wid = _wid()
    pltpu.sync_copy(dst_hbm.at[pl.ds(wid * RPW, RPW)],
                    dst_v.at[pl.ds(0, RPW)])

    @pl.when(wid < NEXTRA)
    def _():
        pltpu.sync_copy(dst_hbm.at[pl.ds(NW * RPW + wid, 1)],
                        dst_v.at[pl.ds(RPW, 1)])

    pltpu.sync_copy(ones_hbm, ones_v)
    base = s * ROWS_PER_TILE
    pltpu.sync_copy(zeros_hbm, deg_sh.at[pl.ds(base, ROWS_PER_TILE)])
    plsc.subcore_barrier()

    def body(j, carry):
        pltpu.sync_copy(ones_v, deg_sh.at[dst_v.at[j]], add=True)
        return carry

    lax.fori_loop(0, RPW, body, 0)

    @pl.when(wid < NEXTRA)
    def _():
        pltpu.sync_copy(ones_v, deg_sh.at[dst_v.at[RPW]], add=True)

    plsc.subcore_barrier()
    pltpu.sync_copy(
        deg_sh.at[pl.ds(base, ROWS_PER_TILE)],
        out_hbm.at[c, pl.ds(base, ROWS_PER_TILE)],
    )


def _make_sc_agg(d):
    @functools.partial(
        pl.kernel,
        out_type=jax.ShapeDtypeStruct((2, NROWS, d), jnp.float32),
        mesh=_mesh,
        compiler_params=_sc_params,
        scratch_types=[
            pltpu.VMEM((RPW + 1, CHUNK), jnp.int32),
            pltpu.VMEM((RPW + 1, CHUNK), jnp.int32),
            pltpu.VMEM((2, BUFR * CHUNK, d), jnp.float32),
            pltpu.VMEM_SHARED((NROWS, d), jnp.float32),
            pltpu.SemaphoreType.DMA,
            pltpu.SemaphoreType.DMA,
            pltpu.SemaphoreType.DMA,
            pltpu.SemaphoreType.DMA,
        ],
    )
    def sc_agg(src_hbm, dst_hbm, y_hbm, zeros_hbm, out_hbm,
               src_v, dst_v, rows_v, agg_sh, g0, g1, s0, s1):
        c = lax.axis_index("c")
        s = lax.axis_index("s")
        wid = _wid()
        base = s * ROWS_PER_TILE
        gsem = (g0, g1)
        ssem = (s0, s1)
        rb = wid * RPW
        pltpu.sync_copy(src_hbm.at[pl.ds(rb, RPW)], src_v.at[pl.ds(0, RPW)])
        pltpu.sync_copy(dst_hbm.at[pl.ds(rb, RPW)], dst_v.at[pl.ds(0, RPW)])

        @pl.when(wid < NEXTRA)
        def _():
            pltpu.sync_copy(src_hbm.at[pl.ds(NW * RPW + wid, 1)],
                            src_v.at[pl.ds(RPW, 1)])
            pltpu.sync_copy(dst_hbm.at[pl.ds(NW * RPW + wid, 1)],
                            dst_v.at[pl.ds(RPW, 1)])

        def gather(j, b):
            for k in range(BUFR):
                pltpu.async_copy(y_hbm.at[src_v.at[j * BUFR + k]],
                                 rows_v.at[b, pl.ds(k * CHUNK, CHUNK)],
                                 gsem[b])

        def scatter(j, b):
            for k in range(BUFR):
                pltpu.async_copy(rows_v.at[b, pl.ds(k * CHUNK, CHUNK)],
                                 agg_sh.at[dst_v.at[j * BUFR + k]],
                                 ssem[b], add=True)

        gather(0, 0)
        pltpu.sync_copy(zeros_hbm, agg_sh.at[pl.ds(base, ROWS_PER_TILE)])
        plsc.subcore_barrier()

        def body(g2, carry):
            for b in range(2):
                j = g2 * 2 + b
                nb = 1 - b

                @pl.when(j >= 1)
                def _():
                    pltpu.make_async_copy(
                        y_hbm.at[pl.ds(0, BUFR * CHUNK)], rows_v.at[nb],
                        ssem[nb]).wait()

                @pl.when(j + 1 < NITER)
                def _():
                    gather(j + 1, nb)

                pltpu.make_async_copy(
                    y_hbm.at[pl.ds(0, BUFR * CHUNK)], rows_v.at[b],
                    gsem[b]).wait()
                scatter(j, b)
            return carry

        lax.fori_loop(0, NITER // 2, body, 0)
        pltpu.make_async_copy(
            y_hbm.at[pl.ds(0, BUFR * CHUNK)], rows_v.at[1], ssem[1]).wait()

        @pl.when(wid < NEXTRA)
        def _():
            pltpu.async_copy(y_hbm.at[src_v.at[RPW]],
                             rows_v.at[0, pl.ds(0, CHUNK)], g0)
            pltpu.make_async_copy(
                y_hbm.at[pl.ds(0, CHUNK)],
                rows_v.at[0, pl.ds(0, CHUNK)], g0).wait()
            pltpu.async_copy(rows_v.at[0, pl.ds(0, CHUNK)],
                             agg_sh.at[dst_v.at[RPW]], s0, add=True)
            pltpu.make_async_copy(
                y_hbm.at[pl.ds(0, CHUNK)],
                rows_v.at[0, pl.ds(0, CHUNK)], s0).wait()

        plsc.subcore_barrier()
        pltpu.sync_copy(
            agg_sh.at[pl.ds(base, ROWS_PER_TILE)],
            out_hbm.at[c, pl.ds(base, ROWS_PER_TILE)],
        )

    return sc_agg


_sc_agg64 = _make_sc_agg(D_HID)
_sc_agg16 = _make_sc_agg(D_OUT)


def _dis_from(degp_ref):
    deg = degp_ref[0] + degp_ref[1] + 2.0
    return lax.rsqrt(deg)[:N, 0:1]


def _psum10k(aggp_ref):
    return aggp_ref[0, :N] + aggp_ref[1, :N]


def _tcmm_body(x_ref, w1_ref, xw_ref):
    xw_ref[...] = jnp.dot(x_ref[...], w1_ref[...],
                          preferred_element_type=jnp.float32)


def _tc1_body(xw_ref, degp_ref, y1_ref):
    y1_ref[...] = xw_ref[...] * _dis_from(degp_ref)


def _tc2_body(aggp_ref, y1_ref, degp_ref, w2_ref, b1_ref, y2_ref):
    dis = _dis_from(degp_ref)
    pre = (_psum10k(aggp_ref) + 2.0 * y1_ref[...]) * dis + b1_ref[...]
    h = jnp.maximum(pre, 0.0)
    y2_ref[...] = jnp.dot(h, w2_ref[...], preferred_element_type=jnp.float32) * dis


def _tc3_body(aggp_ref, y2_ref, degp_ref, b2_ref, out_ref):
    dis = _dis_from(degp_ref)
    o = (_psum10k(aggp_ref) + 2.0 * y2_ref[...]) * dis + b2_ref[...]
    m = jnp.max(o, axis=1, keepdims=True)
    e = jnp.exp(o - m)
    lse = jnp.log(jnp.sum(e, axis=1, keepdims=True))
    out_ref[...] = o - m - lse


_tcmm = pl.pallas_call(
    _tcmm_body,
    out_shape=jax.ShapeDtypeStruct((N, D_HID), jnp.float32),
)

_tc1 = pl.pallas_call(
    _tc1_body,
    out_shape=jax.ShapeDtypeStruct((N, D_HID), jnp.float32),
)

_tc2 = pl.pallas_call(
    _tc2_body,
    out_shape=jax.ShapeDtypeStruct((N, D_OUT), jnp.float32),
)

_tc3 = pl.pallas_call(
    _tc3_body,
    out_shape=jax.ShapeDtypeStruct((N, D_OUT), jnp.float32),
)


def kernel(x, edge_index, W1, b1, W2, b2):
    ei = edge_index.astype(jnp.int32)
    src2d = ei[0].reshape(NCH, CHUNK)
    dst2d = ei[1].reshape(NCH, CHUNK)

    ones_deg = jnp.ones((CHUNK, DEG_W), jnp.float32)
    zeros_deg = jnp.zeros((ROWS_PER_TILE, DEG_W), jnp.float32)
    zeros64 = jnp.zeros((ROWS_PER_TILE, D_HID), jnp.float32)
    zeros16 = jnp.zeros((ROWS_PER_TILE, D_OUT), jnp.float32)

    xw = _tcmm(x, W1)
    degp = _sc_degree(dst2d, ones_deg, zeros_deg)
    y1 = _tc1(xw, degp)
    agg1 = _sc_agg64(src2d, dst2d, y1, zeros64)
    y2 = _tc2(agg1, y1, degp, W2, b1.reshape(1, D_HID))
    agg2 = _sc_agg16(src2d, dst2d, y2, zeros16)
    return _tc3(agg2, y2, degp, b2.reshape(1, D_OUT))
